# baseline (speedup 1.0000x reference)
import functools
import math

import jax
import jax.numpy as jnp
from jax import lax
from jax.experimental import pallas as pl
from jax.experimental.pallas import tpu as pltpu
from jax.experimental.pallas import tpu_sc as plsc

N_NODES = 10000
N_EDGES = 160000
IN_CH = 48
HID = 128
NR = 6
NS = 7
CUTOFF = 5.0
NGRAPH = 16

NPAD = 10240
EPAD = 163840
CH = 128
DUMMY_NODE = 10200

F32 = jnp.float32


def _sc_gather(table, idx):
    mesh = plsc.VectorSubcoreMesh(core_axis_name="c", subcore_axis_name="s")
    per_w = EPAD // 32
    nchunk = per_w // CH

    @functools.partial(
        pl.kernel,
        out_type=jax.ShapeDtypeStruct((EPAD, HID), F32),
        mesh=mesh,
        scratch_types=[
            pltpu.VMEM((CH,), jnp.int32),
            pltpu.VMEM((CH, HID), F32),
            pltpu.SemaphoreType.DMA,
        ],
    )
    def k(table_hbm, idx_hbm, out_hbm, idx_v, rows_v, sem):
        wid = lax.axis_index("s") * 2 + lax.axis_index("c")
        base = wid * per_w

        def body(t, carry):
            off = base + t * CH
            pltpu.sync_copy(idx_hbm.at[pl.ds(off, CH)], idx_v)
            pltpu.async_copy(table_hbm.at[idx_v], rows_v, sem).wait()
            pltpu.sync_copy(rows_v, out_hbm.at[pl.ds(off, CH)])
            return carry

        lax.fori_loop(0, nchunk, body, 0)

    return k(table, idx)


def _sc_scatter_add(msg, idx, zeros):
    mesh = plsc.VectorSubcoreMesh(core_axis_name="c", subcore_axis_name="s")
    per_w = EPAD // 32
    nchunk = per_w // CH
    rows_per_s = NPAD // 16

    @functools.partial(
        pl.kernel,
        out_type=jax.ShapeDtypeStruct((2, NPAD, HID), F32),
        mesh=mesh,
        scratch_types=[
            pltpu.VMEM((CH,), jnp.int32),
            pltpu.VMEM((CH, HID), F32),
            pltpu.VMEM_SHARED((NPAD, HID), F32),
        ],
    )
    def k(msg_hbm, idx_hbm, zeros_hbm, out_hbm, idx_v, buf_v, acc_sh):
        cid = lax.axis_index("c")
        sid = lax.axis_index("s")
        r0 = sid * rows_per_s
        pltpu.sync_copy(zeros_hbm.at[pl.ds(r0, rows_per_s)],
                        acc_sh.at[pl.ds(r0, rows_per_s)])
        plsc.subcore_barrier()

        base = (sid * 2 + cid) * per_w

        def body(t, carry):
            off = base + t * CH
            pltpu.sync_copy(idx_hbm.at[pl.ds(off, CH)], idx_v)
            pltpu.sync_copy(msg_hbm.at[pl.ds(off, CH)], buf_v)
            pltpu.sync_copy(buf_v, acc_sh.at[idx_v], add=True)
            return carry

        lax.fori_loop(0, nchunk, body, 0)
        plsc.subcore_barrier()
        pltpu.sync_copy(acc_sh.at[pl.ds(r0, rows_per_s)],
                        out_hbm.at[cid, pl.ds(r0, rows_per_s)])

    return k(msg, idx, zeros)


def _dot(a, b):
    return jnp.dot(a, b, preferred_element_type=F32)


def _embed_kernel(x_ref, w_ref, b_ref, o_ref):
    o_ref[...] = jax.nn.silu(_dot(x_ref[...], w_ref[...]) + b_ref[...])


def _embed(x_p, w, b):
    bn = 2048
    return pl.pallas_call(
        _embed_kernel,
        grid=(NPAD // bn,),
        in_specs=[
            pl.BlockSpec((bn, IN_CH), lambda n: (n, 0)),
            pl.BlockSpec((IN_CH, HID), lambda n: (0, 0)),
            pl.BlockSpec((1, HID), lambda n: (0, 0)),
        ],
        out_specs=pl.BlockSpec((bn, HID), lambda n: (n, 0)),
        out_shape=jax.ShapeDtypeStruct((NPAD, HID), F32),
    )(x_p, w, b)


def _edge_kernel(rbf_ref, tbf_ref, sbf_ref, hj_ref,
                 w1r, b1r, w2r, b2r,
                 w1t, b1t, w2t, b2t,
                 w1s, b1s, w2s, b2s,
                 mr_ref, mt_ref, ms_ref):
    h = hj_ref[...]

    def mlp2(xb, w1, b1, w2, b2):
        t = jax.nn.silu(_dot(xb, w1[...]) + b1[...])
        return jax.nn.silu(_dot(t, w2[...]) + b2[...])

    mr_ref[...] = jax.nn.silu(h * mlp2(rbf_ref[...], w1r, b1r, w2r, b2r))
    mt_ref[...] = jax.nn.silu(h * mlp2(tbf_ref[...], w1t, b1t, w2t, b2t))
    ms_ref[...] = jax.nn.silu(h * mlp2(sbf_ref[...], w1s, b1s, w2s, b2s))


def _edge_messages(rbf_p, tbf_p, sbf_p, hj, lp):
    be = 2048
    w1r, b1r = lp['lin_rbf'][0]
    w2r, b2r = lp['lin_rbf'][1]
    w1t, b1t = lp['lin_t'][0]
    w2t, b2t = lp['lin_t'][1]
    w1s, b1s = lp['lin_s'][0]
    w2s, b2s = lp['lin_s'][1]

    def wspec(shape):
        return pl.BlockSpec(shape, lambda e: (0, 0))

    out = pl.pallas_call(
        _edge_kernel,
        grid=(EPAD // be,),
        in_specs=[
            pl.BlockSpec((be, 9), lambda e: (e, 0)),
            pl.BlockSpec((be, NS * NS * NR), lambda e: (e, 0)),
            pl.BlockSpec((be, NS * NR), lambda e: (e, 0)),
            pl.BlockSpec((be, HID), lambda e: (e, 0)),
            wspec((9, HID)), wspec((1, HID)), wspec((HID, HID)), wspec((1, HID)),
            wspec((NS * NS * NR, HID)), wspec((1, HID)), wspec((HID, HID)), wspec((1, HID)),
            wspec((NS * NR, HID)), wspec((1, HID)), wspec((HID, HID)), wspec((1, HID)),
        ],
        out_specs=[pl.BlockSpec((be, HID), lambda e: (e, 0))] * 3,
        out_shape=[jax.ShapeDtypeStruct((EPAD, HID), F32)] * 3,
    )(rbf_p, tbf_p, sbf_p, hj,
      w1r, b1r.reshape(1, HID), w2r, b2r.reshape(1, HID),
      w1t, b1t.reshape(1, HID), w2t, b2t.reshape(1, HID),
      w1s, b1s.reshape(1, HID), w2s, b2s.reshape(1, HID))
    return out


def _node_kernel(pr_ref, pt_ref, ps_ref, h_ref,
                 wl1, bl1, wl2, bl2, wl3, bl3,
                 wo1, bo1, g1, be1,
                 wo2, bo2, g2, be2,
                 wo3, bo3, g3, be3,
                 wcat, bcat, out_ref):
    hh = h_ref[...]
    bn_scale = jnp.sqrt(jnp.float32(1.0 + 1e-5))

    def branch(p_ref, wl, bl, wo, bo, g, be_):
        agg = p_ref[0] + p_ref[1]
        h1 = jax.nn.silu(_dot(agg + hh, wl[...]) + bl[...])
        y = _dot(h1, wo[...]) + bo[...]
        y = jnp.where(y >= 0, y, 0.01 * y)
        return y / bn_scale * g[...] + be_[...]

    o1 = branch(pr_ref, wl1, bl1, wo1, bo1, g1, be1)
    o2 = branch(pt_ref, wl2, bl2, wo2, bo2, g2, be2)
    o3 = branch(ps_ref, wl3, bl3, wo3, bo3, g3, be3)
    cat = jnp.concatenate([o1, o2, o3], axis=1)
    out_ref[...] = _dot(cat, wcat[...]) + bcat[...]


def _node_update(pr, pt, ps, h, lp):
    bn = 2048

    def pspec():
        return pl.BlockSpec((2, bn, HID), lambda n: (0, n, 0))

    def wspec(shape):
        return pl.BlockSpec(shape, lambda n: (0, 0))

    wl1, bl1 = lp['lin1']
    wl2, bl2 = lp['lin2']
    wl3, bl3 = lp['lin3']
    wo1, bo1 = lp['out1']['lin']
    wo2, bo2 = lp['out2']['lin']
    wo3, bo3 = lp['out3']['lin']
    wcat, bcat = lp['lin_cat']

    return pl.pallas_call(
        _node_kernel,
        grid=(NPAD // bn,),
        in_specs=[
            pspec(), pspec(), pspec(),
            pl.BlockSpec((bn, HID), lambda n: (n, 0)),
            wspec((HID, HID)), wspec((1, HID)),
            wspec((HID, HID)), wspec((1, HID)),
            wspec((HID, HID)), wspec((1, HID)),
            wspec((HID, HID)), wspec((1, HID)), wspec((1, HID)), wspec((1, HID)),
            wspec((HID, HID)), wspec((1, HID)), wspec((1, HID)), wspec((1, HID)),
            wspec((HID, HID)), wspec((1, HID)), wspec((1, HID)), wspec((1, HID)),
            wspec((3 * HID, HID)), wspec((1, HID)),
        ],
        out_specs=pl.BlockSpec((bn, HID), lambda n: (n, 0)),
        out_shape=jax.ShapeDtypeStruct((NPAD, HID), F32),
    )(pr, pt, ps, h,
      wl1, bl1.reshape(1, HID), wl2, bl2.reshape(1, HID), wl3, bl3.reshape(1, HID),
      wo1, bo1.reshape(1, HID), lp['out1']['gamma'].reshape(1, HID), lp['out1']['beta'].reshape(1, HID),
      wo2, bo2.reshape(1, HID), lp['out2']['gamma'].reshape(1, HID), lp['out2']['beta'].reshape(1, HID),
      wo3, bo3.reshape(1, HID), lp['out3']['gamma'].reshape(1, HID), lp['out3']['beta'].reshape(1, HID),
      wcat, bcat.reshape(1, HID))


def _head_kernel(h_ref, w1, b1, w2, b2, wout, bout, oh_ref, out_ref):
    hh = jax.nn.silu(_dot(h_ref[...], w1[...]) + b1[...])
    hh = jax.nn.silu(_dot(hh, w2[...]) + b2[...])
    s = _dot(hh, wout[...]) + bout[...]
    part = lax.dot_general(oh_ref[...], s, (((0,), (0,)), ((), ())),
                           preferred_element_type=F32)

    @pl.when(pl.program_id(0) == 0)
    def _():
        out_ref[...] = jnp.zeros_like(out_ref)

    out_ref[...] += part


def _head(h, params, onehot_p):
    bn = 2048
    (w1, b1), (w2, b2) = params['lins']
    wout, bout = params['lin_out']

    def wspec(shape):
        return pl.BlockSpec(shape, lambda n: (0, 0))

    return pl.pallas_call(
        _head_kernel,
        grid=(NPAD // bn,),
        in_specs=[
            pl.BlockSpec((bn, HID), lambda n: (n, 0)),
            wspec((HID, HID)), wspec((1, HID)),
            wspec((HID, HID)), wspec((1, HID)),
            wspec((HID, 1)), wspec((1, 1)),
            pl.BlockSpec((bn, NGRAPH), lambda n: (n, 0)),
        ],
        out_specs=pl.BlockSpec((NGRAPH, 1), lambda n: (0, 0)),
        out_shape=jax.ShapeDtypeStruct((NGRAPH, 1), F32),
    )(h, w1, b1.reshape(1, HID), w2, b2.reshape(1, HID),
      wout, bout.reshape(1, 1), onehot_p)


def _rbf9(d):
    mu = jnp.linspace(0.0, 6.0, 9)[None, :]
    sigma = 6.0 / 9.0
    return jnp.exp(-(((d[:, None] - mu) / sigma) ** 2))


def _radial(dist):
    d = dist / CUTOFF
    n = jnp.arange(1, NR + 1, dtype=F32)
    return jnp.sin(n * math.pi * d[:, None]) / (d[:, None] + 1e-8)


def _angle_emb(dist, angle):
    rb = _radial(dist)
    cb = jnp.cos(jnp.arange(NS, dtype=F32) * angle[:, None])
    return (cb[:, :, None] * rb[:, None, :]).reshape(dist.shape[0], NS * NR)


def _torsion_emb(dist, theta, phi):
    rb = _radial(dist)
    ct = jnp.cos(jnp.arange(NS, dtype=F32) * theta[:, None])
    cp = jnp.cos(jnp.arange(NS, dtype=F32) * phi[:, None])
    o = ct[:, :, None, None] * cp[:, None, :, None] * rb[:, None, None, :]
    return o.reshape(dist.shape[0], NS * NS * NR)


def _geometry(pos, edge_index):
    N = pos.shape[0]
    E = edge_index.shape[1]
    j = edge_index[0]
    i = edge_index[1]
    pp = pos[jnp.concatenate([j, i])]
    vecs = pp[:E] - pp[E:]
    dist = jnp.linalg.norm(vecs, axis=-1)
    rbf_feat = _rbf9(dist)

    seg = jnp.concatenate([i, j + N])
    e2 = jnp.tile(jnp.arange(E), 2)
    vals2 = jnp.concatenate([dist, dist])
    mm = jax.ops.segment_min(vals2, seg, num_segments=2 * N)
    arg0 = jax.ops.segment_min(jnp.where(vals2 == mm[seg], e2, E), seg,
                               num_segments=2 * N)
    arg0 = jnp.where(arg0 >= E, 0, arg0)
    argmin0, argmin0_j = arg0[:N], arg0[N:]
    idx_cat = jnp.concatenate([argmin0, argmin0_j + E])
    add2 = jnp.zeros((2 * E,), F32).at[idx_cat].set(CUTOFF)
    vals2b = vals2 + add2
    mm2 = jax.ops.segment_min(vals2b, seg, num_segments=2 * N)
    arg1 = jax.ops.segment_min(jnp.where(vals2b == mm2[seg], e2, E), seg,
                               num_segments=2 * N)
    arg1 = jnp.where(arg1 >= E, 0, arg1)
    argmin1, argmin1_j = arg1[:N], arg1[N:]

    ji = jnp.concatenate([j, i])
    n0cat = ji[idx_cat]
    n0, n0_j = n0cat[:N], n0cat[N:]

    tbl_i = jnp.stack([n0, argmin0, argmin1], axis=1)
    tbl_j = jnp.stack([n0_j, argmin0_j, argmin1_j], axis=1)
    gi = tbl_i[i]
    gj = tbl_j[j]
    n0e, argmin0_i, argmin1_i = gi[:, 0], gi[:, 1], gi[:, 2]
    n0je, argmin0_je, argmin1_je = gj[:, 0], gj[:, 1], gj[:, 2]
    idx_iref = jnp.where(n0e == j, argmin1_i, argmin0_i)
    idx_jref = jnp.where(n0je == i, argmin1_je, argmin0_je)

    idxE4 = jnp.concatenate([argmin0_i, argmin1_i, idx_iref, idx_jref])
    v4 = vecs[idxE4]
    pos_ji = vecs
    pos_in0 = v4[:E]
    pos_in1 = v4[E:2 * E]
    pos_iref = v4[2 * E:3 * E]
    pos_jref = v4[3 * E:]
    a = (-pos_ji * pos_in0).sum(-1)
    b = jnp.linalg.norm(jnp.cross(-pos_ji, pos_in0), axis=-1)
    theta = jnp.arctan2(b, a)
    theta = jnp.where(theta < 0, theta + math.pi, theta)
    dist_ji = jnp.sqrt((pos_ji ** 2).sum(-1))
    p1 = jnp.cross(-pos_ji, pos_in0)
    p2 = jnp.cross(-pos_ji, pos_in1)
    a = (p1 * p2).sum(-1)
    b = (jnp.cross(p1, p2) * pos_ji).sum(-1) / (dist_ji + 1e-12)
    phi = jnp.arctan2(b, a)
    phi = jnp.where(phi < 0, phi + math.pi, phi)
    p1 = jnp.cross(pos_ji, pos_jref)
    p2 = jnp.cross(pos_ji, pos_iref)
    a = (p1 * p2).sum(-1)
    b = (jnp.cross(p1, p2) * pos_ji).sum(-1) / (dist_ji + 1e-12)
    tau = jnp.arctan2(b, a)
    tau = jnp.where(tau < 0, tau + math.pi, tau)
    tbf = _torsion_emb(dist, theta, phi)
    sbf = _angle_emb(dist, tau)
    return rbf_feat, tbf, sbf


def kernel(x, pos, edge_attr, params, edge_index, batch):
    j = edge_index[0]
    i = edge_index[1]
    rbf_feat, tbf, sbf = _geometry(pos, edge_index)

    epad = EPAD - N_EDGES
    j_pad = jnp.concatenate([j, jnp.zeros((epad,), j.dtype)]).astype(jnp.int32)
    i_pad = jnp.concatenate(
        [i, jnp.full((epad,), DUMMY_NODE, i.dtype)]).astype(jnp.int32)
    rbf_p = jnp.pad(rbf_feat, ((0, epad), (0, 0)))
    tbf_p = jnp.pad(tbf, ((0, epad), (0, 0)))
    sbf_p = jnp.pad(sbf, ((0, epad), (0, 0)))
    x_p = jnp.pad(x, ((0, NPAD - N_NODES), (0, 0)))
    onehot = (batch[:, None] == jnp.arange(NGRAPH)[None, :]).astype(F32)
    onehot_p = jnp.pad(onehot, ((0, NPAD - N_NODES), (0, 0)))
    zeros = jnp.zeros((NPAD, HID), F32)

    w_emb, b_emb = params['line_node']
    h = _embed(x_p, w_emb, b_emb.reshape(1, HID))

    for lp in params['layers']:
        hj = _sc_gather(h, j_pad)
        mr, mt, ms = _edge_messages(rbf_p, tbf_p, sbf_p, hj, lp)
        pr = _sc_scatter_add(mr, i_pad, zeros)
        pt = _sc_scatter_add(mt, i_pad, zeros)
        ps = _sc_scatter_add(ms, i_pad, zeros)
        h = _node_update(pr, pt, ps, h, lp)

    return _head(h, params, onehot_p)


# speedup vs baseline: 1.0242x; 1.0242x over previous
import functools
import math

import jax
import jax.numpy as jnp
from jax import lax
from jax.experimental import pallas as pl
from jax.experimental.pallas import tpu as pltpu
from jax.experimental.pallas import tpu_sc as plsc

N_NODES = 10000
N_EDGES = 160000
IN_CH = 48
HID = 128
NR = 6
NS = 7
CUTOFF = 5.0
NGRAPH = 16

NPAD = 10240
EPAD = 163840
CH = 128
DUMMY_NODE = 10200

F32 = jnp.float32


def _sc_gather(table, idx):
    mesh = plsc.VectorSubcoreMesh(core_axis_name="c", subcore_axis_name="s")
    per_w = EPAD // 32
    nchunk = per_w // CH

    @functools.partial(
        pl.kernel,
        out_type=jax.ShapeDtypeStruct((EPAD, HID), F32),
        mesh=mesh,
        scratch_types=[
            pltpu.VMEM((CH,), jnp.int32),
            pltpu.VMEM((CH, HID), F32),
            pltpu.SemaphoreType.DMA,
        ],
    )
    def k(table_hbm, idx_hbm, out_hbm, idx_v, rows_v, sem):
        wid = lax.axis_index("s") * 2 + lax.axis_index("c")
        base = wid * per_w

        def body(t, carry):
            off = base + t * CH
            pltpu.sync_copy(idx_hbm.at[pl.ds(off, CH)], idx_v)
            pltpu.async_copy(table_hbm.at[idx_v], rows_v, sem).wait()
            pltpu.sync_copy(rows_v, out_hbm.at[pl.ds(off, CH)])
            return carry

        lax.fori_loop(0, nchunk, body, 0)

    return k(table, idx)


def _sc_scatter_add(msg, idx, zeros):
    mesh = plsc.VectorSubcoreMesh(core_axis_name="c", subcore_axis_name="s")
    per_w = EPAD // 32
    nchunk = per_w // CH
    rows_per_s = NPAD // 16

    @functools.partial(
        pl.kernel,
        out_type=jax.ShapeDtypeStruct((2, NPAD, HID), F32),
        mesh=mesh,
        scratch_types=[
            pltpu.VMEM((CH,), jnp.int32),
            pltpu.VMEM((CH, HID), F32),
            pltpu.VMEM_SHARED((NPAD, HID), F32),
        ],
    )
    def k(msg_hbm, idx_hbm, zeros_hbm, out_hbm, idx_v, buf_v, acc_sh):
        cid = lax.axis_index("c")
        sid = lax.axis_index("s")
        r0 = sid * rows_per_s
        pltpu.sync_copy(zeros_hbm.at[pl.ds(r0, rows_per_s)],
                        acc_sh.at[pl.ds(r0, rows_per_s)])
        plsc.subcore_barrier()

        base = (sid * 2 + cid) * per_w

        def body(t, carry):
            off = base + t * CH
            pltpu.sync_copy(idx_hbm.at[pl.ds(off, CH)], idx_v)
            pltpu.sync_copy(msg_hbm.at[pl.ds(off, CH)], buf_v)
            pltpu.sync_copy(buf_v, acc_sh.at[idx_v], add=True)
            return carry

        lax.fori_loop(0, nchunk, body, 0)
        plsc.subcore_barrier()
        pltpu.sync_copy(acc_sh.at[pl.ds(r0, rows_per_s)],
                        out_hbm.at[cid, pl.ds(r0, rows_per_s)])

    return k(msg, idx, zeros)


def _dot(a, b):
    return jnp.dot(a, b, preferred_element_type=F32)


def _embed_kernel(x_ref, w_ref, b_ref, o_ref):
    o_ref[...] = jax.nn.silu(_dot(x_ref[...], w_ref[...]) + b_ref[...])


def _embed(x_p, w, b):
    bn = 2048
    return pl.pallas_call(
        _embed_kernel,
        grid=(NPAD // bn,),
        in_specs=[
            pl.BlockSpec((bn, IN_CH), lambda n: (n, 0)),
            pl.BlockSpec((IN_CH, HID), lambda n: (0, 0)),
            pl.BlockSpec((1, HID), lambda n: (0, 0)),
        ],
        out_specs=pl.BlockSpec((bn, HID), lambda n: (n, 0)),
        out_shape=jax.ShapeDtypeStruct((NPAD, HID), F32),
    )(x_p, w, b)


def _edge_kernel(rbf_ref, tbf_ref, sbf_ref, hj_ref,
                 w1r, b1r, w2r, b2r,
                 w1t, b1t, w2t, b2t,
                 w1s, b1s, w2s, b2s,
                 mr_ref, mt_ref, ms_ref):
    h = hj_ref[...]

    def mlp2(xb, w1, b1, w2, b2):
        t = jax.nn.silu(_dot(xb, w1[...]) + b1[...])
        return jax.nn.silu(_dot(t, w2[...]) + b2[...])

    mr_ref[...] = jax.nn.silu(h * mlp2(rbf_ref[...], w1r, b1r, w2r, b2r))
    mt_ref[...] = jax.nn.silu(h * mlp2(tbf_ref[...], w1t, b1t, w2t, b2t))
    ms_ref[...] = jax.nn.silu(h * mlp2(sbf_ref[...], w1s, b1s, w2s, b2s))


def _edge_messages(rbf_p, tbf_p, sbf_p, hj, lp):
    be = 2048
    w1r, b1r = lp['lin_rbf'][0]
    w2r, b2r = lp['lin_rbf'][1]
    w1t, b1t = lp['lin_t'][0]
    w2t, b2t = lp['lin_t'][1]
    w1s, b1s = lp['lin_s'][0]
    w2s, b2s = lp['lin_s'][1]

    def wspec(shape):
        return pl.BlockSpec(shape, lambda e: (0, 0))

    out = pl.pallas_call(
        _edge_kernel,
        grid=(EPAD // be,),
        in_specs=[
            pl.BlockSpec((be, 9), lambda e: (e, 0)),
            pl.BlockSpec((be, NS * NS * NR), lambda e: (e, 0)),
            pl.BlockSpec((be, NS * NR), lambda e: (e, 0)),
            pl.BlockSpec((be, HID), lambda e: (e, 0)),
            wspec((9, HID)), wspec((1, HID)), wspec((HID, HID)), wspec((1, HID)),
            wspec((NS * NS * NR, HID)), wspec((1, HID)), wspec((HID, HID)), wspec((1, HID)),
            wspec((NS * NR, HID)), wspec((1, HID)), wspec((HID, HID)), wspec((1, HID)),
        ],
        out_specs=[pl.BlockSpec((be, HID), lambda e: (e, 0))] * 3,
        out_shape=[jax.ShapeDtypeStruct((EPAD, HID), F32)] * 3,
    )(rbf_p, tbf_p, sbf_p, hj,
      w1r, b1r.reshape(1, HID), w2r, b2r.reshape(1, HID),
      w1t, b1t.reshape(1, HID), w2t, b2t.reshape(1, HID),
      w1s, b1s.reshape(1, HID), w2s, b2s.reshape(1, HID))
    return out


def _node_kernel(pr_ref, pt_ref, ps_ref, h_ref,
                 wl1, bl1, wl2, bl2, wl3, bl3,
                 wo1, bo1, g1, be1,
                 wo2, bo2, g2, be2,
                 wo3, bo3, g3, be3,
                 wcat, bcat, out_ref):
    hh = h_ref[...]
    bn_scale = jnp.sqrt(jnp.float32(1.0 + 1e-5))

    def branch(p_ref, wl, bl, wo, bo, g, be_):
        agg = p_ref[0] + p_ref[1]
        h1 = jax.nn.silu(_dot(agg + hh, wl[...]) + bl[...])
        y = _dot(h1, wo[...]) + bo[...]
        y = jnp.where(y >= 0, y, 0.01 * y)
        return y / bn_scale * g[...] + be_[...]

    o1 = branch(pr_ref, wl1, bl1, wo1, bo1, g1, be1)
    o2 = branch(pt_ref, wl2, bl2, wo2, bo2, g2, be2)
    o3 = branch(ps_ref, wl3, bl3, wo3, bo3, g3, be3)
    cat = jnp.concatenate([o1, o2, o3], axis=1)
    out_ref[...] = _dot(cat, wcat[...]) + bcat[...]


def _node_update(pr, pt, ps, h, lp):
    bn = 2048

    def pspec():
        return pl.BlockSpec((2, bn, HID), lambda n: (0, n, 0))

    def wspec(shape):
        return pl.BlockSpec(shape, lambda n: (0, 0))

    wl1, bl1 = lp['lin1']
    wl2, bl2 = lp['lin2']
    wl3, bl3 = lp['lin3']
    wo1, bo1 = lp['out1']['lin']
    wo2, bo2 = lp['out2']['lin']
    wo3, bo3 = lp['out3']['lin']
    wcat, bcat = lp['lin_cat']

    return pl.pallas_call(
        _node_kernel,
        grid=(NPAD // bn,),
        in_specs=[
            pspec(), pspec(), pspec(),
            pl.BlockSpec((bn, HID), lambda n: (n, 0)),
            wspec((HID, HID)), wspec((1, HID)),
            wspec((HID, HID)), wspec((1, HID)),
            wspec((HID, HID)), wspec((1, HID)),
            wspec((HID, HID)), wspec((1, HID)), wspec((1, HID)), wspec((1, HID)),
            wspec((HID, HID)), wspec((1, HID)), wspec((1, HID)), wspec((1, HID)),
            wspec((HID, HID)), wspec((1, HID)), wspec((1, HID)), wspec((1, HID)),
            wspec((3 * HID, HID)), wspec((1, HID)),
        ],
        out_specs=pl.BlockSpec((bn, HID), lambda n: (n, 0)),
        out_shape=jax.ShapeDtypeStruct((NPAD, HID), F32),
    )(pr, pt, ps, h,
      wl1, bl1.reshape(1, HID), wl2, bl2.reshape(1, HID), wl3, bl3.reshape(1, HID),
      wo1, bo1.reshape(1, HID), lp['out1']['gamma'].reshape(1, HID), lp['out1']['beta'].reshape(1, HID),
      wo2, bo2.reshape(1, HID), lp['out2']['gamma'].reshape(1, HID), lp['out2']['beta'].reshape(1, HID),
      wo3, bo3.reshape(1, HID), lp['out3']['gamma'].reshape(1, HID), lp['out3']['beta'].reshape(1, HID),
      wcat, bcat.reshape(1, HID))


def _head_kernel(h_ref, w1, b1, w2, b2, wout, bout, oh_ref, out_ref):
    hh = jax.nn.silu(_dot(h_ref[...], w1[...]) + b1[...])
    hh = jax.nn.silu(_dot(hh, w2[...]) + b2[...])
    s = _dot(hh, wout[...]) + bout[...]
    part = lax.dot_general(oh_ref[...], s, (((0,), (0,)), ((), ())),
                           preferred_element_type=F32)

    @pl.when(pl.program_id(0) == 0)
    def _():
        out_ref[...] = jnp.zeros_like(out_ref)

    out_ref[...] += part


def _head(h, params, onehot_p):
    bn = 2048
    (w1, b1), (w2, b2) = params['lins']
    wout, bout = params['lin_out']

    def wspec(shape):
        return pl.BlockSpec(shape, lambda n: (0, 0))

    return pl.pallas_call(
        _head_kernel,
        grid=(NPAD // bn,),
        in_specs=[
            pl.BlockSpec((bn, HID), lambda n: (n, 0)),
            wspec((HID, HID)), wspec((1, HID)),
            wspec((HID, HID)), wspec((1, HID)),
            wspec((HID, 1)), wspec((1, 1)),
            pl.BlockSpec((bn, NGRAPH), lambda n: (n, 0)),
        ],
        out_specs=pl.BlockSpec((NGRAPH, 1), lambda n: (0, 0)),
        out_shape=jax.ShapeDtypeStruct((NGRAPH, 1), F32),
    )(h, w1, b1.reshape(1, HID), w2, b2.reshape(1, HID),
      wout, bout.reshape(1, 1), onehot_p)


def _rbf9(d):
    mu = jnp.linspace(0.0, 6.0, 9)[None, :]
    sigma = 6.0 / 9.0
    return jnp.exp(-(((d[:, None] - mu) / sigma) ** 2))


def _radial(dist):
    d = dist / CUTOFF
    n = jnp.arange(1, NR + 1, dtype=F32)
    return jnp.sin(n * math.pi * d[:, None]) / (d[:, None] + 1e-8)


def _angle_emb(dist, angle):
    rb = _radial(dist)
    cb = jnp.cos(jnp.arange(NS, dtype=F32) * angle[:, None])
    return (cb[:, :, None] * rb[:, None, :]).reshape(dist.shape[0], NS * NR)


def _torsion_emb(dist, theta, phi):
    rb = _radial(dist)
    ct = jnp.cos(jnp.arange(NS, dtype=F32) * theta[:, None])
    cp = jnp.cos(jnp.arange(NS, dtype=F32) * phi[:, None])
    o = ct[:, :, None, None] * cp[:, None, :, None] * rb[:, None, None, :]
    return o.reshape(dist.shape[0], NS * NS * NR)


def _geometry(pos, edge_index):
    N = pos.shape[0]
    E = edge_index.shape[1]
    j = edge_index[0]
    i = edge_index[1]
    pp = pos[jnp.concatenate([j, i])]
    vecs = pp[:E] - pp[E:]
    dist = jnp.linalg.norm(vecs, axis=-1)
    rbf_feat = _rbf9(dist)

    def sam(vals, idx):
        minv = jax.ops.segment_min(vals, idx, num_segments=N)
        ismin = vals == minv[idx]
        arg = jax.ops.segment_min(jnp.where(ismin, jnp.arange(E), E), idx,
                                  num_segments=N)
        return jnp.where(arg >= E, 0, arg)

    argmin0 = sam(dist, i)
    add = jnp.zeros((E,), F32).at[argmin0].set(CUTOFF)
    argmin1 = sam(dist + add, i)
    argmin0_j = sam(dist, j)
    add_j = jnp.zeros((E,), F32).at[argmin0_j].set(CUTOFF)
    argmin1_j = sam(dist + add_j, j)

    ji = jnp.concatenate([j, i])
    n0cat = ji[jnp.concatenate([argmin0, argmin0_j + E])]
    n0, n0_j = n0cat[:N], n0cat[N:]

    tbl_i = jnp.stack([n0, argmin0, argmin1], axis=1)
    tbl_j = jnp.stack([n0_j, argmin0_j, argmin1_j], axis=1)
    gi = tbl_i[i]
    gj = tbl_j[j]
    n0e, argmin0_i, argmin1_i = gi[:, 0], gi[:, 1], gi[:, 2]
    n0je, argmin0_je, argmin1_je = gj[:, 0], gj[:, 1], gj[:, 2]
    idx_iref = jnp.where(n0e == j, argmin1_i, argmin0_i)
    idx_jref = jnp.where(n0je == i, argmin1_je, argmin0_je)

    idxE4 = jnp.concatenate([argmin0_i, argmin1_i, idx_iref, idx_jref])
    v4 = vecs[idxE4]
    pos_ji = vecs
    pos_in0 = v4[:E]
    pos_in1 = v4[E:2 * E]
    pos_iref = v4[2 * E:3 * E]
    pos_jref = v4[3 * E:]
    a = (-pos_ji * pos_in0).sum(-1)
    b = jnp.linalg.norm(jnp.cross(-pos_ji, pos_in0), axis=-1)
    theta = jnp.arctan2(b, a)
    theta = jnp.where(theta < 0, theta + math.pi, theta)
    dist_ji = jnp.sqrt((pos_ji ** 2).sum(-1))
    p1 = jnp.cross(-pos_ji, pos_in0)
    p2 = jnp.cross(-pos_ji, pos_in1)
    a = (p1 * p2).sum(-1)
    b = (jnp.cross(p1, p2) * pos_ji).sum(-1) / (dist_ji + 1e-12)
    phi = jnp.arctan2(b, a)
    phi = jnp.where(phi < 0, phi + math.pi, phi)
    p1 = jnp.cross(pos_ji, pos_jref)
    p2 = jnp.cross(pos_ji, pos_iref)
    a = (p1 * p2).sum(-1)
    b = (jnp.cross(p1, p2) * pos_ji).sum(-1) / (dist_ji + 1e-12)
    tau = jnp.arctan2(b, a)
    tau = jnp.where(tau < 0, tau + math.pi, tau)
    tbf = _torsion_emb(dist, theta, phi)
    sbf = _angle_emb(dist, tau)
    return rbf_feat, tbf, sbf


def kernel(x, pos, edge_attr, params, edge_index, batch):
    j = edge_index[0]
    i = edge_index[1]
    rbf_feat, tbf, sbf = _geometry(pos, edge_index)

    epad = EPAD - N_EDGES
    j_pad = jnp.concatenate([j, jnp.zeros((epad,), j.dtype)]).astype(jnp.int32)
    i_pad = jnp.concatenate(
        [i, jnp.full((epad,), DUMMY_NODE, i.dtype)]).astype(jnp.int32)
    rbf_p = jnp.pad(rbf_feat, ((0, epad), (0, 0)))
    tbf_p = jnp.pad(tbf, ((0, epad), (0, 0)))
    sbf_p = jnp.pad(sbf, ((0, epad), (0, 0)))
    x_p = jnp.pad(x, ((0, NPAD - N_NODES), (0, 0)))
    onehot = (batch[:, None] == jnp.arange(NGRAPH)[None, :]).astype(F32)
    onehot_p = jnp.pad(onehot, ((0, NPAD - N_NODES), (0, 0)))
    zeros = jnp.zeros((NPAD, HID), F32)

    w_emb, b_emb = params['line_node']
    h = _embed(x_p, w_emb, b_emb.reshape(1, HID))

    for lp in params['layers']:
        hj = _sc_gather(h, j_pad)
        mr, mt, ms = _edge_messages(rbf_p, tbf_p, sbf_p, hj, lp)
        pr = _sc_scatter_add(mr, i_pad, zeros)
        pt = _sc_scatter_add(mt, i_pad, zeros)
        ps = _sc_scatter_add(ms, i_pad, zeros)
        h = _node_update(pr, pt, ps, h, lp)

    return _head(h, params, onehot_p)


# speedup vs baseline: 1.3687x; 1.3363x over previous
import functools
import math

import jax
import jax.numpy as jnp
from jax import lax
from jax.experimental import pallas as pl
from jax.experimental.pallas import tpu as pltpu
from jax.experimental.pallas import tpu_sc as plsc

N_NODES = 10000
N_EDGES = 160000
IN_CH = 48
HID = 128
NR = 6
NS = 7
CUTOFF = 5.0
NGRAPH = 16

NPAD = 10240
EPAD = 163840
CH = 128
DUMMY_NODE = 10200

F32 = jnp.float32


def _sc_gather(table, idx):
    mesh = plsc.VectorSubcoreMesh(core_axis_name="c", subcore_axis_name="s")
    per_w = EPAD // 32
    nchunk = per_w // CH

    @functools.partial(
        pl.kernel,
        out_type=jax.ShapeDtypeStruct((EPAD, HID), F32),
        mesh=mesh,
        scratch_types=[
            pltpu.VMEM((CH,), jnp.int32),
            pltpu.VMEM((CH, HID), F32),
            pltpu.SemaphoreType.DMA,
        ],
    )
    def k(table_hbm, idx_hbm, out_hbm, idx_v, rows_v, sem):
        wid = lax.axis_index("s") * 2 + lax.axis_index("c")
        base = wid * per_w

        def body(t, carry):
            off = base + t * CH
            pltpu.sync_copy(idx_hbm.at[pl.ds(off, CH)], idx_v)
            pltpu.async_copy(table_hbm.at[idx_v], rows_v, sem).wait()
            pltpu.sync_copy(rows_v, out_hbm.at[pl.ds(off, CH)])
            return carry

        lax.fori_loop(0, nchunk, body, 0)

    return k(table, idx)


def _sc_scatter_add(msg, idx, zeros):
    mesh = plsc.VectorSubcoreMesh(core_axis_name="c", subcore_axis_name="s")
    per_w = EPAD // 32
    nchunk = per_w // CH
    rows_per_s = NPAD // 16

    @functools.partial(
        pl.kernel,
        out_type=jax.ShapeDtypeStruct((2, NPAD, HID), F32),
        mesh=mesh,
        scratch_types=[
            pltpu.VMEM((CH,), jnp.int32),
            pltpu.VMEM((CH, HID), F32),
            pltpu.VMEM_SHARED((NPAD, HID), F32),
        ],
    )
    def k(msg_hbm, idx_hbm, zeros_hbm, out_hbm, idx_v, buf_v, acc_sh):
        cid = lax.axis_index("c")
        sid = lax.axis_index("s")
        r0 = sid * rows_per_s
        pltpu.sync_copy(zeros_hbm.at[pl.ds(r0, rows_per_s)],
                        acc_sh.at[pl.ds(r0, rows_per_s)])
        plsc.subcore_barrier()

        base = (sid * 2 + cid) * per_w

        def body(t, carry):
            off = base + t * CH
            pltpu.sync_copy(idx_hbm.at[pl.ds(off, CH)], idx_v)
            pltpu.sync_copy(msg_hbm.at[pl.ds(off, CH)], buf_v)
            pltpu.sync_copy(buf_v, acc_sh.at[idx_v], add=True)
            return carry

        lax.fori_loop(0, nchunk, body, 0)
        plsc.subcore_barrier()
        pltpu.sync_copy(acc_sh.at[pl.ds(r0, rows_per_s)],
                        out_hbm.at[cid, pl.ds(r0, rows_per_s)])

    return k(msg, idx, zeros)


def _dot(a, b):
    return jnp.dot(a, b, preferred_element_type=F32)


def _embed_kernel(x_ref, w_ref, b_ref, o_ref):
    o_ref[...] = jax.nn.silu(_dot(x_ref[...], w_ref[...]) + b_ref[...])


def _embed(x_p, w, b):
    bn = 2048
    return pl.pallas_call(
        _embed_kernel,
        grid=(NPAD // bn,),
        in_specs=[
            pl.BlockSpec((bn, IN_CH), lambda n: (n, 0)),
            pl.BlockSpec((IN_CH, HID), lambda n: (0, 0)),
            pl.BlockSpec((1, HID), lambda n: (0, 0)),
        ],
        out_specs=pl.BlockSpec((bn, HID), lambda n: (n, 0)),
        out_shape=jax.ShapeDtypeStruct((NPAD, HID), F32),
    )(x_p, w, b)


def _edge_kernel(rbf_ref, tbf_ref, sbf_ref, hj_ref,
                 w1r, b1r, w2r, b2r,
                 w1t, b1t, w2t, b2t,
                 w1s, b1s, w2s, b2s,
                 mr_ref, mt_ref, ms_ref):
    h = hj_ref[...]

    def mlp2(xb, w1, b1, w2, b2):
        t = jax.nn.silu(_dot(xb, w1[...]) + b1[...])
        return jax.nn.silu(_dot(t, w2[...]) + b2[...])

    mr_ref[...] = jax.nn.silu(h * mlp2(rbf_ref[...], w1r, b1r, w2r, b2r))
    mt_ref[...] = jax.nn.silu(h * mlp2(tbf_ref[...], w1t, b1t, w2t, b2t))
    ms_ref[...] = jax.nn.silu(h * mlp2(sbf_ref[...], w1s, b1s, w2s, b2s))


def _edge_messages(rbf_p, tbf_p, sbf_p, hj, lp):
    be = 2048
    w1r, b1r = lp['lin_rbf'][0]
    w2r, b2r = lp['lin_rbf'][1]
    w1t, b1t = lp['lin_t'][0]
    w2t, b2t = lp['lin_t'][1]
    w1s, b1s = lp['lin_s'][0]
    w2s, b2s = lp['lin_s'][1]

    def wspec(shape):
        return pl.BlockSpec(shape, lambda e: (0, 0))

    out = pl.pallas_call(
        _edge_kernel,
        grid=(EPAD // be,),
        in_specs=[
            pl.BlockSpec((be, 9), lambda e: (e, 0)),
            pl.BlockSpec((be, NS * NS * NR), lambda e: (e, 0)),
            pl.BlockSpec((be, NS * NR), lambda e: (e, 0)),
            pl.BlockSpec((be, HID), lambda e: (e, 0)),
            wspec((9, HID)), wspec((1, HID)), wspec((HID, HID)), wspec((1, HID)),
            wspec((NS * NS * NR, HID)), wspec((1, HID)), wspec((HID, HID)), wspec((1, HID)),
            wspec((NS * NR, HID)), wspec((1, HID)), wspec((HID, HID)), wspec((1, HID)),
        ],
        out_specs=[pl.BlockSpec((be, HID), lambda e: (e, 0))] * 3,
        out_shape=[jax.ShapeDtypeStruct((EPAD, HID), F32)] * 3,
    )(rbf_p, tbf_p, sbf_p, hj,
      w1r, b1r.reshape(1, HID), w2r, b2r.reshape(1, HID),
      w1t, b1t.reshape(1, HID), w2t, b2t.reshape(1, HID),
      w1s, b1s.reshape(1, HID), w2s, b2s.reshape(1, HID))
    return out


def _node_kernel(pr_ref, pt_ref, ps_ref, h_ref,
                 wl1, bl1, wl2, bl2, wl3, bl3,
                 wo1, bo1, g1, be1,
                 wo2, bo2, g2, be2,
                 wo3, bo3, g3, be3,
                 wcat, bcat, out_ref):
    hh = h_ref[...]
    bn_scale = jnp.sqrt(jnp.float32(1.0 + 1e-5))

    def branch(p_ref, wl, bl, wo, bo, g, be_):
        agg = p_ref[0] + p_ref[1]
        h1 = jax.nn.silu(_dot(agg + hh, wl[...]) + bl[...])
        y = _dot(h1, wo[...]) + bo[...]
        y = jnp.where(y >= 0, y, 0.01 * y)
        return y / bn_scale * g[...] + be_[...]

    o1 = branch(pr_ref, wl1, bl1, wo1, bo1, g1, be1)
    o2 = branch(pt_ref, wl2, bl2, wo2, bo2, g2, be2)
    o3 = branch(ps_ref, wl3, bl3, wo3, bo3, g3, be3)
    cat = jnp.concatenate([o1, o2, o3], axis=1)
    out_ref[...] = _dot(cat, wcat[...]) + bcat[...]


def _node_update(pr, pt, ps, h, lp):
    bn = 2048

    def pspec():
        return pl.BlockSpec((2, bn, HID), lambda n: (0, n, 0))

    def wspec(shape):
        return pl.BlockSpec(shape, lambda n: (0, 0))

    wl1, bl1 = lp['lin1']
    wl2, bl2 = lp['lin2']
    wl3, bl3 = lp['lin3']
    wo1, bo1 = lp['out1']['lin']
    wo2, bo2 = lp['out2']['lin']
    wo3, bo3 = lp['out3']['lin']
    wcat, bcat = lp['lin_cat']

    return pl.pallas_call(
        _node_kernel,
        grid=(NPAD // bn,),
        in_specs=[
            pspec(), pspec(), pspec(),
            pl.BlockSpec((bn, HID), lambda n: (n, 0)),
            wspec((HID, HID)), wspec((1, HID)),
            wspec((HID, HID)), wspec((1, HID)),
            wspec((HID, HID)), wspec((1, HID)),
            wspec((HID, HID)), wspec((1, HID)), wspec((1, HID)), wspec((1, HID)),
            wspec((HID, HID)), wspec((1, HID)), wspec((1, HID)), wspec((1, HID)),
            wspec((HID, HID)), wspec((1, HID)), wspec((1, HID)), wspec((1, HID)),
            wspec((3 * HID, HID)), wspec((1, HID)),
        ],
        out_specs=pl.BlockSpec((bn, HID), lambda n: (n, 0)),
        out_shape=jax.ShapeDtypeStruct((NPAD, HID), F32),
    )(pr, pt, ps, h,
      wl1, bl1.reshape(1, HID), wl2, bl2.reshape(1, HID), wl3, bl3.reshape(1, HID),
      wo1, bo1.reshape(1, HID), lp['out1']['gamma'].reshape(1, HID), lp['out1']['beta'].reshape(1, HID),
      wo2, bo2.reshape(1, HID), lp['out2']['gamma'].reshape(1, HID), lp['out2']['beta'].reshape(1, HID),
      wo3, bo3.reshape(1, HID), lp['out3']['gamma'].reshape(1, HID), lp['out3']['beta'].reshape(1, HID),
      wcat, bcat.reshape(1, HID))


def _head_kernel(h_ref, w1, b1, w2, b2, wout, bout, oh_ref, out_ref):
    hh = jax.nn.silu(_dot(h_ref[...], w1[...]) + b1[...])
    hh = jax.nn.silu(_dot(hh, w2[...]) + b2[...])
    s = _dot(hh, wout[...]) + bout[...]
    part = lax.dot_general(oh_ref[...], s, (((0,), (0,)), ((), ())),
                           preferred_element_type=F32)

    @pl.when(pl.program_id(0) == 0)
    def _():
        out_ref[...] = jnp.zeros_like(out_ref)

    out_ref[...] += part


def _head(h, params, onehot_p):
    bn = 2048
    (w1, b1), (w2, b2) = params['lins']
    wout, bout = params['lin_out']

    def wspec(shape):
        return pl.BlockSpec(shape, lambda n: (0, 0))

    return pl.pallas_call(
        _head_kernel,
        grid=(NPAD // bn,),
        in_specs=[
            pl.BlockSpec((bn, HID), lambda n: (n, 0)),
            wspec((HID, HID)), wspec((1, HID)),
            wspec((HID, HID)), wspec((1, HID)),
            wspec((HID, 1)), wspec((1, 1)),
            pl.BlockSpec((bn, NGRAPH), lambda n: (n, 0)),
        ],
        out_specs=pl.BlockSpec((NGRAPH, 1), lambda n: (0, 0)),
        out_shape=jax.ShapeDtypeStruct((NGRAPH, 1), F32),
    )(h, w1, b1.reshape(1, HID), w2, b2.reshape(1, HID),
      wout, bout.reshape(1, 1), onehot_p)


def _rbf9(d):
    mu = jnp.linspace(0.0, 6.0, 9)[None, :]
    sigma = 6.0 / 9.0
    return jnp.exp(-(((d[:, None] - mu) / sigma) ** 2))


def _radial(dist):
    d = dist / CUTOFF
    n = jnp.arange(1, NR + 1, dtype=F32)
    return jnp.sin(n * math.pi * d[:, None]) / (d[:, None] + 1e-8)


def _angle_emb(dist, angle):
    rb = _radial(dist)
    cb = jnp.cos(jnp.arange(NS, dtype=F32) * angle[:, None])
    return (cb[:, :, None] * rb[:, None, :]).reshape(dist.shape[0], NS * NR)


def _torsion_emb(dist, theta, phi):
    rb = _radial(dist)
    ct = jnp.cos(jnp.arange(NS, dtype=F32) * theta[:, None])
    cp = jnp.cos(jnp.arange(NS, dtype=F32) * phi[:, None])
    o = ct[:, :, None, None] * cp[:, None, :, None] * rb[:, None, None, :]
    return o.reshape(dist.shape[0], NS * NS * NR)


def _geometry(pos, edge_index):
    N = pos.shape[0]
    E = edge_index.shape[1]
    j = edge_index[0]
    i = edge_index[1]
    vecs = pos[j] - pos[i]
    dist = jnp.linalg.norm(vecs, axis=-1)
    rbf_feat = _rbf9(dist)
    eids = jnp.arange(E)

    def sam(vals, idx):
        minv = jax.ops.segment_min(vals, idx, num_segments=N)
        ismin = vals == minv[idx]
        arg = jax.ops.segment_min(jnp.where(ismin, eids, E), idx,
                                  num_segments=N)
        return jnp.where(arg >= E, 0, arg), arg >= E

    argmin0, empty_i = sam(dist, i)
    argmin0_i = argmin0[i]
    add = jnp.where((argmin0_i == eids) | (jnp.any(empty_i) & (eids == 0)),
                    CUTOFF, 0.0)
    argmin1, _ = sam(dist + add, i)
    argmin0_j, empty_j = sam(dist, j)
    argmin0_je = argmin0_j[j]
    add_j = jnp.where((argmin0_je == eids) | (jnp.any(empty_j) & (eids == 0)),
                      CUTOFF, 0.0)
    argmin1_j, _ = sam(dist + add_j, j)
    n0 = j[argmin0]
    n0_j = i[argmin0_j]
    n0e = n0[i]
    n0je = n0_j[j]
    idx_iref = jnp.where(n0e == j, argmin1[i], argmin0_i)
    idx_jref = jnp.where(n0je == i, argmin1_j[j], argmin0_je)
    pos_ji = vecs
    pos_in0 = vecs[argmin0][i]
    pos_in1 = vecs[argmin1][i]
    pos_iref = vecs[idx_iref]
    pos_jref = vecs[idx_jref]
    a = (-pos_ji * pos_in0).sum(-1)
    b = jnp.linalg.norm(jnp.cross(-pos_ji, pos_in0), axis=-1)
    theta = jnp.arctan2(b, a)
    theta = jnp.where(theta < 0, theta + math.pi, theta)
    dist_ji = jnp.sqrt((pos_ji ** 2).sum(-1))
    p1 = jnp.cross(-pos_ji, pos_in0)
    p2 = jnp.cross(-pos_ji, pos_in1)
    a = (p1 * p2).sum(-1)
    b = (jnp.cross(p1, p2) * pos_ji).sum(-1) / (dist_ji + 1e-12)
    phi = jnp.arctan2(b, a)
    phi = jnp.where(phi < 0, phi + math.pi, phi)
    p1 = jnp.cross(pos_ji, pos_jref)
    p2 = jnp.cross(pos_ji, pos_iref)
    a = (p1 * p2).sum(-1)
    b = (jnp.cross(p1, p2) * pos_ji).sum(-1) / (dist_ji + 1e-12)
    tau = jnp.arctan2(b, a)
    tau = jnp.where(tau < 0, tau + math.pi, tau)
    tbf = _torsion_emb(dist, theta, phi)
    sbf = _angle_emb(dist, tau)
    return rbf_feat, tbf, sbf


def kernel(x, pos, edge_attr, params, edge_index, batch):
    j = edge_index[0]
    i = edge_index[1]
    rbf_feat, tbf, sbf = _geometry(pos, edge_index)

    epad = EPAD - N_EDGES
    j_pad = jnp.concatenate([j, jnp.zeros((epad,), j.dtype)]).astype(jnp.int32)
    i_pad = jnp.concatenate(
        [i, jnp.full((epad,), DUMMY_NODE, i.dtype)]).astype(jnp.int32)
    rbf_p = jnp.pad(rbf_feat, ((0, epad), (0, 0)))
    tbf_p = jnp.pad(tbf, ((0, epad), (0, 0)))
    sbf_p = jnp.pad(sbf, ((0, epad), (0, 0)))
    x_p = jnp.pad(x, ((0, NPAD - N_NODES), (0, 0)))
    onehot = (batch[:, None] == jnp.arange(NGRAPH)[None, :]).astype(F32)
    onehot_p = jnp.pad(onehot, ((0, NPAD - N_NODES), (0, 0)))
    zeros = jnp.zeros((NPAD, HID), F32)

    w_emb, b_emb = params['line_node']
    h = _embed(x_p, w_emb, b_emb.reshape(1, HID))

    for lp in params['layers']:
        hj = _sc_gather(h, j_pad)
        mr, mt, ms = _edge_messages(rbf_p, tbf_p, sbf_p, hj, lp)
        pr = _sc_scatter_add(mr, i_pad, zeros)
        pt = _sc_scatter_add(mt, i_pad, zeros)
        ps = _sc_scatter_add(ms, i_pad, zeros)
        h = _node_update(pr, pt, ps, h, lp)

    return _head(h, params, onehot_p)
